# baseline (speedup 1.0000x reference)
import jax
import jax.numpy as jnp
from jax import lax
from jax.experimental import pallas as pl
from jax.experimental.pallas import tpu as pltpu
from jax.experimental.pallas import tpu_sc as plsc

DIM = 64
NCTX = 5
NLANE = 16
NC = 2
NS = 16
HALF = DIM // NC
B = 16384
NLOOK = B * (NCTX + 1)
DUMP = NLOOK
SLOTS = NLOOK + 1

RNG = 62464
WINR = 512
NWIN = RNG // WINR
SUPW = 16
NSUP = 8
TAILR = 512
FINR = 64
RING = 16

CAP_T, CAP_C = 2048, 7168
SCAP_T, SCAP_C = 768, 1536
GCAP_T, GCAP_C = 256, 512
CCHUNK = 64
VOCABF = NS * RNG + TAILR


def _filter(src_r, src_p, cnt, dst_r, dst_p, lo, width, cap):
    lane = lax.iota(jnp.int32, NLANE)

    def body(g, cur):
        rl = src_r[pl.ds(g * NLANE, NLANE)] - lo
        p = src_p[pl.ds(g * NLANE, NLANE)]
        m = (rl >= 0) & (rl < width)
        plsc.store_compressed(dst_r.at[pl.ds(cur, NLANE)], rl, mask=m)
        plsc.store_compressed(dst_p.at[pl.ds(cur, NLANE)], p, mask=m)
        cnt_vec = plsc.all_reduce_population_count(m)
        return jnp.minimum(cur + cnt_vec[0], cap)

    ngroups = lax.div(cnt + NLANE - 1, NLANE)
    out = lax.fori_loop(0, ngroups, body, 0)
    dst_r[pl.ds(out, NLANE)] = jnp.zeros((NLANE,), jnp.int32)
    dst_p[pl.ds(out, NLANE)] = jnp.full((NLANE,), DUMP, jnp.int32)
    del lane
    return out


def _drain_slot(idx_hbm, drain_v, semr, slot):
    pltpu.make_async_copy(
        idx_hbm.at[pl.ds(0, NLANE * NLANE)], drain_v,
        semr.at[slot]).wait()


def _extract(gr, gp, cnt, win, rowring, drain_v, stage_hbm, c, idx_hbm,
             semr, gctr, drain_all=False):
    lane = lax.iota(jnp.int32, NLANE)

    def body(g, carry):
        tick = gctr + g
        slot = tick % RING

        @pl.when(tick >= RING)
        def _():
            _drain_slot(idx_hbm, drain_v, semr, slot)

        rlv = gr[pl.ds(g * NLANE, NLANE)]
        plv = gp[pl.ds(g * NLANE, NLANE)]
        rbase = slot * NLANE * NLANE
        for l in range(NLANE):
            rs = jnp.full((NLANE,), rlv[l], jnp.int32)
            roff = rbase + l * NLANE
            lo = plsc.load_gather(win, [lane, rs])
            hi = plsc.load_gather(win, [lane + NLANE, rs])
            packed = plsc.pack(lo, hi,
                               format=plsc.PackFormat.INTERLEAVED)
            rowring[pl.ds(roff, NLANE)] = plsc.bitcast(packed, jnp.int32)
            pltpu.async_copy(
                rowring.at[pl.ds(roff, NLANE)],
                stage_hbm.at[pl.ds((c * SLOTS + plv[l]) * NLANE, NLANE)],
                semr.at[slot])
        return carry

    n = lax.div(cnt + NLANE - 1, NLANE)
    lax.fori_loop(0, n, body, 0)
    if drain_all:
        def dbody(j, carry):
            _drain_slot(idx_hbm, drain_v, semr, (gctr + n - 1 - j) % RING)
            return carry

        lax.fori_loop(0, jnp.minimum(n, RING), dbody, 0)
        return gctr
    return gctr + n


def _sc_body(idx_all_hbm, tgt_tab_hbm, ctx_tab_hbm, tail_t_hbm, tail_c_hbm,
             out_hbm, stage_hbm,
             idx_v, wl_tr, wl_tp, wl_cr, wl_cp, swl_r, swl_p,
             gwl_r, gwl_p, win_t0, win_t1, win_c0, win_c1, win_ft, win_fc,
             rowring, drain_v, tbuf, cbuf, out_v, semr, wsem0, wsem1):
    c = lax.axis_index("c")
    s = lax.axis_index("s")
    lane = lax.iota(jnp.int32, NLANE)
    r0 = s * RNG

    def scan_chunk(ci, wr, wp, cap, cur0):
        pltpu.sync_copy(idx_all_hbm.at[pl.ds(ci * 8192, 8192)], idx_v)

        def body(v, cur):
            rl = idx_v[pl.ds(v * NLANE, NLANE)] - r0
            width = jnp.where(s == NS - 1, RNG + TAILR + FINR, RNG)
            m = (rl >= 0) & (rl < width)
            plsc.store_compressed(wr.at[pl.ds(cur, NLANE)], rl, mask=m)
            p = ci * 8192 + v * NLANE + lane
            plsc.store_compressed(wp.at[pl.ds(cur, NLANE)], p, mask=m)
            cnt_vec = plsc.all_reduce_population_count(m)
            return jnp.minimum(cur + cnt_vec[0], cap)

        return lax.fori_loop(0, 8192 // NLANE, body, cur0)

    cnt_t = 0
    for ci in range(2):
        cnt_t = scan_chunk(ci, wl_tr, wl_tp, CAP_T, cnt_t)
    cnt_c = 0
    for ci in range(2, 12):
        cnt_c = scan_chunk(ci, wl_cr, wl_cp, CAP_C, cnt_c)
    wl_tr[pl.ds(cnt_t, NLANE)] = jnp.zeros((NLANE,), jnp.int32)
    wl_tp[pl.ds(cnt_t, NLANE)] = jnp.full((NLANE,), DUMP, jnp.int32)
    wl_cr[pl.ds(cnt_c, NLANE)] = jnp.zeros((NLANE,), jnp.int32)
    wl_cp[pl.ds(cnt_c, NLANE)] = jnp.full((NLANE,), DUMP, jnp.int32)

    wins = ((win_t0, win_c0, wsem0), (win_t1, win_c1, wsem1))

    def start_win(rw, par):
        wt, wc, wsem = wins[par]
        pltpu.async_copy(
            tgt_tab_hbm.at[pl.ds(c * HALF, HALF), pl.ds(rw, WINR)],
            wt, wsem)
        pltpu.async_copy(
            ctx_tab_hbm.at[pl.ds(c * HALF, HALF), pl.ds(rw, WINR)],
            wc, wsem)

    def wait_win(par):
        wt, wc, wsem = wins[par]
        pltpu.make_async_copy(
            tgt_tab_hbm.at[pl.ds(0, HALF), pl.ds(0, WINR)], wt,
            wsem).wait()
        pltpu.make_async_copy(
            tgt_tab_hbm.at[pl.ds(0, HALF), pl.ds(0, WINR)], wc,
            wsem).wait()

    def super_body(sp, gctr0):
        lo_sp = sp * (SUPW * WINR)
        nw = jnp.where(sp == NSUP - 1, NWIN - (NSUP - 1) * SUPW, SUPW)
        sc_t = _filter(wl_tr, wl_tp, cnt_t, swl_r, swl_p, lo_sp,
                       nw * WINR, SCAP_T)
        sc_c = _filter(wl_cr, wl_cp, cnt_c,
                       swl_r.at[pl.ds(SCAP_T + NLANE, SCAP_C + NLANE)],
                       swl_p.at[pl.ds(SCAP_T + NLANE, SCAP_C + NLANE)],
                       lo_sp, nw * WINR, SCAP_C)
        start_win(r0 + lo_sp, 0)
        start_win(r0 + lo_sp + WINR, 1)

        def pair_body(gpair, gctr1):
            gctr2 = gctr1
            for par in range(2):
                w = gpair * 2 + par
                rw = r0 + lo_sp + w * WINR
                wt, wc, _ = wins[par]
                wait_win(par)
                g_t = _filter(swl_r, swl_p, sc_t, gwl_r, gwl_p,
                              w * WINR, WINR, GCAP_T)
                gctr2 = _extract(gwl_r, gwl_p, g_t, wt, rowring, drain_v,
                                 stage_hbm, c, idx_all_hbm, semr, gctr2)
                g_c = _filter(
                    swl_r.at[pl.ds(SCAP_T + NLANE, SCAP_C + NLANE)],
                    swl_p.at[pl.ds(SCAP_T + NLANE, SCAP_C + NLANE)],
                    sc_c,
                    gwl_r.at[pl.ds(GCAP_T + NLANE, GCAP_C + NLANE)],
                    gwl_p.at[pl.ds(GCAP_T + NLANE, GCAP_C + NLANE)],
                    w * WINR, WINR, GCAP_C)
                gctr2 = _extract(
                    gwl_r.at[pl.ds(GCAP_T + NLANE, GCAP_C + NLANE)],
                    gwl_p.at[pl.ds(GCAP_T + NLANE, GCAP_C + NLANE)],
                    g_c, wc, rowring, drain_v, stage_hbm, c, idx_all_hbm,
                    semr, gctr2)

                @pl.when(w + 2 < nw)
                def _():
                    start_win(rw + 2 * WINR, par)

            return gctr2

        return lax.fori_loop(0, nw // 2, pair_body, gctr0)

    gctr = lax.fori_loop(0, NSUP, super_body, 0)

    def rbody(j, carry):
        @pl.when(j < jnp.minimum(gctr, RING))
        def _():
            _drain_slot(idx_all_hbm, drain_v, semr, (gctr - 1 - j) % RING)
        return carry

    lax.fori_loop(0, RING, rbody, 0)

    @pl.when(s == NS - 1)
    def _():
        rw = (NS - 1) * RNG + NWIN * WINR
        pltpu.sync_copy(
            tgt_tab_hbm.at[pl.ds(c * HALF, HALF), pl.ds(rw, TAILR)],
            win_t0)
        pltpu.sync_copy(
            ctx_tab_hbm.at[pl.ds(c * HALF, HALF), pl.ds(rw, TAILR)],
            win_c0)
        g_t = _filter(wl_tr, wl_tp, cnt_t, gwl_r, gwl_p,
                      NWIN * WINR, TAILR, GCAP_T)
        _extract(gwl_r, gwl_p, g_t, win_t0, rowring, drain_v,
                 stage_hbm, c, idx_all_hbm, semr, 0, drain_all=True)
        g_c = _filter(wl_cr, wl_cp, cnt_c,
                      gwl_r.at[pl.ds(GCAP_T + NLANE, GCAP_C + NLANE)],
                      gwl_p.at[pl.ds(GCAP_T + NLANE, GCAP_C + NLANE)],
                      NWIN * WINR, TAILR, GCAP_C)
        _extract(gwl_r.at[pl.ds(GCAP_T + NLANE, GCAP_C + NLANE)],
                 gwl_p.at[pl.ds(GCAP_T + NLANE, GCAP_C + NLANE)],
                 g_c, win_c0, rowring, drain_v, stage_hbm, c, idx_all_hbm,
                 semr, 0, drain_all=True)
        pltpu.sync_copy(tail_t_hbm.at[pl.ds(c * HALF, HALF)], win_ft)
        pltpu.sync_copy(tail_c_hbm.at[pl.ds(c * HALF, HALF)], win_fc)
        f_t = _filter(wl_tr, wl_tp, cnt_t, gwl_r, gwl_p,
                      NWIN * WINR + TAILR, FINR, GCAP_T)
        _extract(gwl_r, gwl_p, f_t, win_ft, rowring, drain_v,
                 stage_hbm, c, idx_all_hbm, semr, 0, drain_all=True)
        f_c = _filter(wl_cr, wl_cp, cnt_c,
                      gwl_r.at[pl.ds(GCAP_T + NLANE, GCAP_C + NLANE)],
                      gwl_p.at[pl.ds(GCAP_T + NLANE, GCAP_C + NLANE)],
                      NWIN * WINR + TAILR, FINR, GCAP_C)
        _extract(gwl_r.at[pl.ds(GCAP_T + NLANE, GCAP_C + NLANE)],
                 gwl_p.at[pl.ds(GCAP_T + NLANE, GCAP_C + NLANE)],
                 f_c, win_fc, rowring, drain_v, stage_hbm, c, idx_all_hbm,
                 semr, 0, drain_all=True)

    plsc.subcore_barrier()

    def sub_body(sub, carry):
        bbase = s * 1024 + sub * CCHUNK
        pltpu.sync_copy(
            stage_hbm.at[pl.ds((c * SLOTS + bbase) * NLANE,
                               CCHUNK * NLANE)], tbuf)
        pltpu.sync_copy(
            stage_hbm.at[pl.ds((c * SLOTS + B + bbase * NCTX) * NLANE,
                               CCHUNK * NCTX * NLANE)],
            cbuf)

        def b_body(b0, carry2):
            t0, t1 = plsc.unpack(
                plsc.bitcast(tbuf[pl.ds(b0 * NLANE, NLANE)],
                             jnp.bfloat16),
                format=plsc.PackFormat.INTERLEAVED)
            vals = jnp.zeros((NLANE,), jnp.float32)
            for cc in range(NCTX):
                coff = (b0 * NCTX + cc) * NLANE
                c0, c1 = plsc.unpack(
                    plsc.bitcast(cbuf[pl.ds(coff, NLANE)], jnp.bfloat16),
                    format=plsc.PackFormat.INTERLEAVED)
                acc = c0 * t0
                acc = acc + c1 * t1
                vals = jnp.where(lane == cc, jnp.sum(acc), vals)
            cur = out_v[pl.ds(b0 * NCTX, NLANE)]
            out_v[pl.ds(b0 * NCTX, NLANE)] = jnp.where(lane < NCTX, vals,
                                                       cur)
            return carry2

        lax.fori_loop(0, CCHUNK, b_body, 0)
        pltpu.sync_copy(
            out_v.at[pl.ds(0, CCHUNK * NCTX)],
            out_hbm.at[pl.ds((c * NS * 1024 + bbase) * NCTX,
                             CCHUNK * NCTX)])
        return carry

    lax.fori_loop(0, 1024 // CCHUNK, sub_body, 0)


def _make_call(batch):
    mesh = plsc.VectorSubcoreMesh(core_axis_name="c", subcore_axis_name="s")
    return pl.kernel(
        _sc_body,
        mesh=mesh,
        out_type=(
            jax.ShapeDtypeStruct((NC * batch * NCTX,), jnp.float32),
            jax.ShapeDtypeStruct((NC * SLOTS * NLANE,), jnp.int32),
        ),
        scratch_types=[
            pltpu.VMEM((8192,), jnp.int32),
            pltpu.VMEM((CAP_T + NLANE,), jnp.int32),
            pltpu.VMEM((CAP_T + NLANE,), jnp.int32),
            pltpu.VMEM((CAP_C + NLANE,), jnp.int32),
            pltpu.VMEM((CAP_C + NLANE,), jnp.int32),
            pltpu.VMEM((SCAP_T + SCAP_C + 2 * NLANE,), jnp.int32),
            pltpu.VMEM((SCAP_T + SCAP_C + 2 * NLANE,), jnp.int32),
            pltpu.VMEM((GCAP_T + GCAP_C + 2 * NLANE,), jnp.int32),
            pltpu.VMEM((GCAP_T + GCAP_C + 2 * NLANE,), jnp.int32),
            pltpu.VMEM((HALF, WINR), jnp.float32),
            pltpu.VMEM((HALF, WINR), jnp.float32),
            pltpu.VMEM((HALF, WINR), jnp.float32),
            pltpu.VMEM((HALF, WINR), jnp.float32),
            pltpu.VMEM((HALF, FINR), jnp.float32),
            pltpu.VMEM((HALF, FINR), jnp.float32),
            pltpu.VMEM((RING * NLANE * NLANE,), jnp.int32),
            pltpu.VMEM((NLANE * NLANE,), jnp.int32),
            pltpu.VMEM((CCHUNK * NLANE,), jnp.int32),
            pltpu.VMEM((CCHUNK * NCTX * NLANE,), jnp.int32),
            pltpu.VMEM((CCHUNK * NCTX + NLANE,), jnp.float32),
            pltpu.SemaphoreType.DMA((RING,)),
            pltpu.SemaphoreType.DMA,
            pltpu.SemaphoreType.DMA,
        ],
        compiler_params=pltpu.CompilerParams(needs_layout_passes=False),
    )


@jax.jit
def kernel(target, context, target_table, context_table):
    batch = target.shape[0]
    idx_all = jnp.concatenate(
        [target.reshape(batch), context.reshape(batch * NCTX)])
    tail_t = target_table[VOCABF:].T
    tail_c = context_table[VOCABF:].T
    out, _ = _make_call(batch)(idx_all, target_table.T, context_table.T,
                               tail_t, tail_c)
    out2 = out.reshape(NC, batch * NCTX)
    return (out2[0] + out2[1]).reshape(batch, NCTX)


# speedup vs baseline: 1.0184x; 1.0184x over previous
import jax
import jax.numpy as jnp
from jax import lax
from jax.experimental import pallas as pl
from jax.experimental.pallas import tpu as pltpu
from jax.experimental.pallas import tpu_sc as plsc

DIM = 64
NCTX = 5
NLANE = 16
NC = 2
NS = 16
HALF = DIM // NC
B = 16384
NLOOK = B * (NCTX + 1)
DUMP = NLOOK
SLOTS = NLOOK + 1

RNG = 62464
WINR = 512
NWIN = RNG // WINR
SUPW = 16
NSUP = 8
TAILR = 512
FINR = 64
RING = 16

CAP_T, CAP_C = 2048, 7168
SCAP_T, SCAP_C = 768, 1536
GCAP_T, GCAP_C = 256, 512
CCHUNK = 128
VOCABF = NS * RNG + TAILR


def _filter(src_r, src_p, cnt, dst_r, dst_p, lo, width, cap):
    lane = lax.iota(jnp.int32, NLANE)

    def body(g, cur):
        rl = src_r[pl.ds(g * NLANE, NLANE)] - lo
        p = src_p[pl.ds(g * NLANE, NLANE)]
        m = (rl >= 0) & (rl < width)
        plsc.store_compressed(dst_r.at[pl.ds(cur, NLANE)], rl, mask=m)
        plsc.store_compressed(dst_p.at[pl.ds(cur, NLANE)], p, mask=m)
        cnt_vec = plsc.all_reduce_population_count(m)
        return jnp.minimum(cur + cnt_vec[0], cap)

    ngroups = lax.div(cnt + NLANE - 1, NLANE)
    out = lax.fori_loop(0, ngroups, body, 0)
    dst_r[pl.ds(out, NLANE)] = jnp.zeros((NLANE,), jnp.int32)
    dst_p[pl.ds(out, NLANE)] = jnp.full((NLANE,), DUMP, jnp.int32)
    del lane
    return out


def _drain_slot(idx_hbm, drain_v, semr, slot):
    pltpu.make_async_copy(
        idx_hbm.at[pl.ds(0, NLANE * NLANE)], drain_v,
        semr.at[slot]).wait()


def _extract(gr, gp, cnt, win, rowring, drain_v, stage_hbm, c, idx_hbm,
             semr, gctr, drain_all=False):
    lane = lax.iota(jnp.int32, NLANE)

    def body(g, carry):
        tick = gctr + g
        slot = tick % RING

        @pl.when(tick >= RING)
        def _():
            _drain_slot(idx_hbm, drain_v, semr, slot)

        rlv = gr[pl.ds(g * NLANE, NLANE)]
        plv = gp[pl.ds(g * NLANE, NLANE)]
        rbase = slot * NLANE * NLANE
        for l in range(NLANE):
            rs = jnp.full((NLANE,), rlv[l], jnp.int32)
            roff = rbase + l * NLANE
            lo = plsc.load_gather(win, [lane, rs])
            hi = plsc.load_gather(win, [lane + NLANE, rs])
            packed = plsc.pack(lo, hi,
                               format=plsc.PackFormat.INTERLEAVED)
            rowring[pl.ds(roff, NLANE)] = plsc.bitcast(packed, jnp.int32)
            pltpu.async_copy(
                rowring.at[pl.ds(roff, NLANE)],
                stage_hbm.at[pl.ds((c * SLOTS + plv[l]) * NLANE, NLANE)],
                semr.at[slot])
        return carry

    n = lax.div(cnt + NLANE - 1, NLANE)
    lax.fori_loop(0, n, body, 0)
    if drain_all:
        def dbody(j, carry):
            _drain_slot(idx_hbm, drain_v, semr, (gctr + n - 1 - j) % RING)
            return carry

        lax.fori_loop(0, jnp.minimum(n, RING), dbody, 0)
        return gctr
    return gctr + n


def _sc_body(idx_all_hbm, tgt_tab_hbm, ctx_tab_hbm, tail_t_hbm, tail_c_hbm,
             out_hbm, stage_hbm,
             idx_v, wl_tr, wl_tp, wl_cr, wl_cp, swl_r, swl_p,
             gwl_r, gwl_p, win_t0, win_t1, win_c0, win_c1, win_ft, win_fc,
             rowring, drain_v, tbuf, cbuf, out_v, semr, wsem0, wsem1):
    c = lax.axis_index("c")
    s = lax.axis_index("s")
    lane = lax.iota(jnp.int32, NLANE)
    r0 = s * RNG

    def scan_chunk(ci, wr, wp, cap, cur0):
        pltpu.sync_copy(idx_all_hbm.at[pl.ds(ci * 8192, 8192)], idx_v)

        def body(v, cur):
            rl = idx_v[pl.ds(v * NLANE, NLANE)] - r0
            width = jnp.where(s == NS - 1, RNG + TAILR + FINR, RNG)
            m = (rl >= 0) & (rl < width)
            plsc.store_compressed(wr.at[pl.ds(cur, NLANE)], rl, mask=m)
            p = ci * 8192 + v * NLANE + lane
            plsc.store_compressed(wp.at[pl.ds(cur, NLANE)], p, mask=m)
            cnt_vec = plsc.all_reduce_population_count(m)
            return jnp.minimum(cur + cnt_vec[0], cap)

        return lax.fori_loop(0, 8192 // NLANE, body, cur0)

    cnt_t = 0
    for ci in range(2):
        cnt_t = scan_chunk(ci, wl_tr, wl_tp, CAP_T, cnt_t)
    cnt_c = 0
    for ci in range(2, 12):
        cnt_c = scan_chunk(ci, wl_cr, wl_cp, CAP_C, cnt_c)
    wl_tr[pl.ds(cnt_t, NLANE)] = jnp.zeros((NLANE,), jnp.int32)
    wl_tp[pl.ds(cnt_t, NLANE)] = jnp.full((NLANE,), DUMP, jnp.int32)
    wl_cr[pl.ds(cnt_c, NLANE)] = jnp.zeros((NLANE,), jnp.int32)
    wl_cp[pl.ds(cnt_c, NLANE)] = jnp.full((NLANE,), DUMP, jnp.int32)

    wins = ((win_t0, win_c0, wsem0), (win_t1, win_c1, wsem1))

    def start_win(rw, par):
        wt, wc, wsem = wins[par]
        pltpu.async_copy(
            tgt_tab_hbm.at[pl.ds(c * HALF, HALF), pl.ds(rw, WINR)],
            wt, wsem)
        pltpu.async_copy(
            ctx_tab_hbm.at[pl.ds(c * HALF, HALF), pl.ds(rw, WINR)],
            wc, wsem)

    def wait_win(par):
        wt, wc, wsem = wins[par]
        pltpu.make_async_copy(
            tgt_tab_hbm.at[pl.ds(0, HALF), pl.ds(0, WINR)], wt,
            wsem).wait()
        pltpu.make_async_copy(
            tgt_tab_hbm.at[pl.ds(0, HALF), pl.ds(0, WINR)], wc,
            wsem).wait()

    def super_body(sp, gctr0):
        lo_sp = sp * (SUPW * WINR)
        nw = jnp.where(sp == NSUP - 1, NWIN - (NSUP - 1) * SUPW, SUPW)
        sc_t = _filter(wl_tr, wl_tp, cnt_t, swl_r, swl_p, lo_sp,
                       nw * WINR, SCAP_T)
        sc_c = _filter(wl_cr, wl_cp, cnt_c,
                       swl_r.at[pl.ds(SCAP_T + NLANE, SCAP_C + NLANE)],
                       swl_p.at[pl.ds(SCAP_T + NLANE, SCAP_C + NLANE)],
                       lo_sp, nw * WINR, SCAP_C)
        start_win(r0 + lo_sp, 0)
        start_win(r0 + lo_sp + WINR, 1)

        def pair_body(gpair, gctr1):
            gctr2 = gctr1
            for par in range(2):
                w = gpair * 2 + par
                rw = r0 + lo_sp + w * WINR
                wt, wc, _ = wins[par]
                wait_win(par)
                g_t = _filter(swl_r, swl_p, sc_t, gwl_r, gwl_p,
                              w * WINR, WINR, GCAP_T)
                gctr2 = _extract(gwl_r, gwl_p, g_t, wt, rowring, drain_v,
                                 stage_hbm, c, idx_all_hbm, semr, gctr2)
                g_c = _filter(
                    swl_r.at[pl.ds(SCAP_T + NLANE, SCAP_C + NLANE)],
                    swl_p.at[pl.ds(SCAP_T + NLANE, SCAP_C + NLANE)],
                    sc_c,
                    gwl_r.at[pl.ds(GCAP_T + NLANE, GCAP_C + NLANE)],
                    gwl_p.at[pl.ds(GCAP_T + NLANE, GCAP_C + NLANE)],
                    w * WINR, WINR, GCAP_C)
                gctr2 = _extract(
                    gwl_r.at[pl.ds(GCAP_T + NLANE, GCAP_C + NLANE)],
                    gwl_p.at[pl.ds(GCAP_T + NLANE, GCAP_C + NLANE)],
                    g_c, wc, rowring, drain_v, stage_hbm, c, idx_all_hbm,
                    semr, gctr2)

                @pl.when(w + 2 < nw)
                def _():
                    start_win(rw + 2 * WINR, par)

            return gctr2

        return lax.fori_loop(0, nw // 2, pair_body, gctr0)

    gctr = lax.fori_loop(0, NSUP, super_body, 0)

    def rbody(j, carry):
        @pl.when(j < jnp.minimum(gctr, RING))
        def _():
            _drain_slot(idx_all_hbm, drain_v, semr, (gctr - 1 - j) % RING)
        return carry

    lax.fori_loop(0, RING, rbody, 0)

    @pl.when(s == NS - 1)
    def _():
        rw = (NS - 1) * RNG + NWIN * WINR
        pltpu.sync_copy(
            tgt_tab_hbm.at[pl.ds(c * HALF, HALF), pl.ds(rw, TAILR)],
            win_t0)
        pltpu.sync_copy(
            ctx_tab_hbm.at[pl.ds(c * HALF, HALF), pl.ds(rw, TAILR)],
            win_c0)
        g_t = _filter(wl_tr, wl_tp, cnt_t, gwl_r, gwl_p,
                      NWIN * WINR, TAILR, GCAP_T)
        _extract(gwl_r, gwl_p, g_t, win_t0, rowring, drain_v,
                 stage_hbm, c, idx_all_hbm, semr, 0, drain_all=True)
        g_c = _filter(wl_cr, wl_cp, cnt_c,
                      gwl_r.at[pl.ds(GCAP_T + NLANE, GCAP_C + NLANE)],
                      gwl_p.at[pl.ds(GCAP_T + NLANE, GCAP_C + NLANE)],
                      NWIN * WINR, TAILR, GCAP_C)
        _extract(gwl_r.at[pl.ds(GCAP_T + NLANE, GCAP_C + NLANE)],
                 gwl_p.at[pl.ds(GCAP_T + NLANE, GCAP_C + NLANE)],
                 g_c, win_c0, rowring, drain_v, stage_hbm, c, idx_all_hbm,
                 semr, 0, drain_all=True)
        pltpu.sync_copy(tail_t_hbm.at[pl.ds(c * HALF, HALF)], win_ft)
        pltpu.sync_copy(tail_c_hbm.at[pl.ds(c * HALF, HALF)], win_fc)
        f_t = _filter(wl_tr, wl_tp, cnt_t, gwl_r, gwl_p,
                      NWIN * WINR + TAILR, FINR, GCAP_T)
        _extract(gwl_r, gwl_p, f_t, win_ft, rowring, drain_v,
                 stage_hbm, c, idx_all_hbm, semr, 0, drain_all=True)
        f_c = _filter(wl_cr, wl_cp, cnt_c,
                      gwl_r.at[pl.ds(GCAP_T + NLANE, GCAP_C + NLANE)],
                      gwl_p.at[pl.ds(GCAP_T + NLANE, GCAP_C + NLANE)],
                      NWIN * WINR + TAILR, FINR, GCAP_C)
        _extract(gwl_r.at[pl.ds(GCAP_T + NLANE, GCAP_C + NLANE)],
                 gwl_p.at[pl.ds(GCAP_T + NLANE, GCAP_C + NLANE)],
                 f_c, win_fc, rowring, drain_v, stage_hbm, c, idx_all_hbm,
                 semr, 0, drain_all=True)

    plsc.subcore_barrier()

    def sub_body(sub, carry):
        bbase = s * 1024 + sub * CCHUNK
        pltpu.sync_copy(
            stage_hbm.at[pl.ds((c * SLOTS + bbase) * NLANE,
                               CCHUNK * NLANE)], tbuf)
        pltpu.sync_copy(
            stage_hbm.at[pl.ds((c * SLOTS + B + bbase * NCTX) * NLANE,
                               CCHUNK * NCTX * NLANE)],
            cbuf)

        def b_body(b0, carry2):
            t0, t1 = plsc.unpack(
                plsc.bitcast(tbuf[pl.ds(b0 * NLANE, NLANE)],
                             jnp.bfloat16),
                format=plsc.PackFormat.INTERLEAVED)
            vals = jnp.zeros((NLANE,), jnp.float32)
            for cc in range(NCTX):
                coff = (b0 * NCTX + cc) * NLANE
                c0, c1 = plsc.unpack(
                    plsc.bitcast(cbuf[pl.ds(coff, NLANE)], jnp.bfloat16),
                    format=plsc.PackFormat.INTERLEAVED)
                acc = c0 * t0
                acc = acc + c1 * t1
                vals = jnp.where(lane == cc, jnp.sum(acc), vals)
            cur = out_v[pl.ds(b0 * NCTX, NLANE)]
            out_v[pl.ds(b0 * NCTX, NLANE)] = jnp.where(lane < NCTX, vals,
                                                       cur)
            return carry2

        lax.fori_loop(0, CCHUNK, b_body, 0)
        pltpu.sync_copy(
            out_v.at[pl.ds(0, CCHUNK * NCTX)],
            out_hbm.at[pl.ds((c * NS * 1024 + bbase) * NCTX,
                             CCHUNK * NCTX)])
        return carry

    lax.fori_loop(0, 1024 // CCHUNK, sub_body, 0)


def _make_call(batch):
    mesh = plsc.VectorSubcoreMesh(core_axis_name="c", subcore_axis_name="s")
    return pl.kernel(
        _sc_body,
        mesh=mesh,
        out_type=(
            jax.ShapeDtypeStruct((NC * batch * NCTX,), jnp.float32),
            jax.ShapeDtypeStruct((NC * SLOTS * NLANE,), jnp.int32),
        ),
        scratch_types=[
            pltpu.VMEM((8192,), jnp.int32),
            pltpu.VMEM((CAP_T + NLANE,), jnp.int32),
            pltpu.VMEM((CAP_T + NLANE,), jnp.int32),
            pltpu.VMEM((CAP_C + NLANE,), jnp.int32),
            pltpu.VMEM((CAP_C + NLANE,), jnp.int32),
            pltpu.VMEM((SCAP_T + SCAP_C + 2 * NLANE,), jnp.int32),
            pltpu.VMEM((SCAP_T + SCAP_C + 2 * NLANE,), jnp.int32),
            pltpu.VMEM((GCAP_T + GCAP_C + 2 * NLANE,), jnp.int32),
            pltpu.VMEM((GCAP_T + GCAP_C + 2 * NLANE,), jnp.int32),
            pltpu.VMEM((HALF, WINR), jnp.float32),
            pltpu.VMEM((HALF, WINR), jnp.float32),
            pltpu.VMEM((HALF, WINR), jnp.float32),
            pltpu.VMEM((HALF, WINR), jnp.float32),
            pltpu.VMEM((HALF, FINR), jnp.float32),
            pltpu.VMEM((HALF, FINR), jnp.float32),
            pltpu.VMEM((RING * NLANE * NLANE,), jnp.int32),
            pltpu.VMEM((NLANE * NLANE,), jnp.int32),
            pltpu.VMEM((CCHUNK * NLANE,), jnp.int32),
            pltpu.VMEM((CCHUNK * NCTX * NLANE,), jnp.int32),
            pltpu.VMEM((CCHUNK * NCTX + NLANE,), jnp.float32),
            pltpu.SemaphoreType.DMA((RING,)),
            pltpu.SemaphoreType.DMA,
            pltpu.SemaphoreType.DMA,
        ],
        compiler_params=pltpu.CompilerParams(needs_layout_passes=False),
    )


@jax.jit
def kernel(target, context, target_table, context_table):
    batch = target.shape[0]
    idx_all = jnp.concatenate(
        [target.reshape(batch), context.reshape(batch * NCTX)])
    tail_t = target_table[VOCABF:].T
    tail_c = context_table[VOCABF:].T
    out, _ = _make_call(batch)(idx_all, target_table.T, context_table.T,
                               tail_t, tail_c)
    out2 = out.reshape(NC, batch * NCTX)
    return (out2[0] + out2[1]).reshape(batch, NCTX)
